# baseline (speedup 1.0000x reference)
import functools

import jax
import jax.numpy as jnp
from jax import lax
from jax.experimental import pallas as pl
from jax.experimental.pallas import tpu as pltpu
from jax.experimental.pallas import tpu_sc as plsc

N_STOCKS = 100000
N_SECTORS = 20
D_MODEL = 64
BATCH = 16384

_NC = 2
_NS = 16
_NW = _NC * _NS
_BPW = BATCH // _NW
_CHUNK = 128
_NCH = _BPW // _CHUNK


def _emb_body(stock_ids_hbm, sector_ids_hbm, stock_tab_hbm, sector_tab_hbm,
              out_hbm, sidx_v, eidx_v, srows_v, erows_v, sem):
    wid = lax.axis_index("s") * _NC + lax.axis_index("c")
    base = wid * _BPW

    pltpu.sync_copy(stock_ids_hbm.at[pl.ds(wid * _NCH, _NCH)], sidx_v)
    pltpu.sync_copy(sector_ids_hbm.at[pl.ds(wid * _NCH, _NCH)], eidx_v)

    copies = []
    for c in range(_NCH):
        copies.append(pltpu.async_copy(
            stock_tab_hbm.at[sidx_v.at[c]],
            srows_v.at[pl.ds(c * _CHUNK, _CHUNK)], sem))
        copies.append(pltpu.async_copy(
            sector_tab_hbm.at[eidx_v.at[c]],
            erows_v.at[pl.ds(c * _CHUNK, _CHUNK)], sem))
    for cp in copies:
        cp.wait()

    def body(b, carry):
        for j in range(D_MODEL // 16):
            sl = pl.ds(j * 16, 16)
            srows_v[b, sl] = srows_v[b, sl] + erows_v[b, sl]
        return carry

    lax.fori_loop(0, _BPW, body, 0)

    pltpu.sync_copy(srows_v, out_hbm.at[pl.ds(base, _BPW)])


def kernel(stock_ids, sector_ids, stock_table, sector_table):
    mesh = plsc.VectorSubcoreMesh(core_axis_name="c", subcore_axis_name="s")
    run = functools.partial(
        pl.kernel,
        mesh=mesh,
        out_type=jax.ShapeDtypeStruct((BATCH, D_MODEL), jnp.float32),
        scratch_types=[
            pltpu.VMEM((_NCH, _CHUNK), jnp.int32),
            pltpu.VMEM((_NCH, _CHUNK), jnp.int32),
            pltpu.VMEM((_BPW, D_MODEL), jnp.float32),
            pltpu.VMEM((_BPW, D_MODEL), jnp.float32),
            pltpu.SemaphoreType.DMA,
        ],
        compiler_params=pltpu.CompilerParams(use_tc_tiling_on_sc=False),
    )(_emb_body)
    sids = stock_ids.astype(jnp.int32).reshape(_NW * _NCH, _CHUNK)
    eids = sector_ids.astype(jnp.int32).reshape(_NW * _NCH, _CHUNK)
    return run(sids, eids, stock_table, sector_table)


# speedup vs baseline: 1.3454x; 1.3454x over previous
import functools

import jax
import jax.numpy as jnp
from jax import lax
from jax.experimental import pallas as pl
from jax.experimental.pallas import tpu as pltpu
from jax.experimental.pallas import tpu_sc as plsc

N_STOCKS = 100000
N_SECTORS = 20
D_MODEL = 64
BATCH = 16384

_NC = 2
_NS = 16
_NW = _NC * _NS
_BPW = BATCH // _NW
_CHUNK = 128
_NCH = _BPW // _CHUNK


def _emb_body(stock_ids_hbm, sector_ids_hbm, stock_tab_hbm, sector_tab_hbm,
              out_hbm, sidx_v, eidx_v, sec_v, srows_v, erows_v,
              isem, gsem, esem, osem):
    wid = lax.axis_index("s") * _NC + lax.axis_index("c")
    base = wid * _BPW

    sub = lax.axis_index("s")

    cp_si = pltpu.async_copy(stock_ids_hbm.at[pl.ds(wid * _NCH, _NCH)],
                             sidx_v, isem)
    cp_ei = pltpu.async_copy(sector_ids_hbm.at[pl.ds(wid * _NCH, _NCH)],
                             eidx_v, isem)

    @pl.when(sub == 0)
    def _():
        pltpu.sync_copy(sector_tab_hbm, sec_v)

    cp_si.wait()
    gcps = [pltpu.async_copy(stock_tab_hbm.at[sidx_v.at[c]],
                             srows_v.at[pl.ds(c * _CHUNK, _CHUNK)], gsem)
            for c in range(_NCH)]
    cp_ei.wait()
    plsc.subcore_barrier()
    ecps = [pltpu.async_copy(sec_v.at[eidx_v.at[c]],
                             erows_v.at[pl.ds(c * _CHUNK, _CHUNK)], esem)
            for c in range(_NCH)]

    ocps = []
    for c in range(_NCH):
        gcps[c].wait()
        ecps[c].wait()

        def body(b, carry):
            for j in range(D_MODEL // 16):
                sl = pl.ds(j * 16, 16)
                srows_v[b, sl] = srows_v[b, sl] + erows_v[b, sl]
            return carry

        lax.fori_loop(c * _CHUNK, (c + 1) * _CHUNK, body, 0)
        ocps.append(pltpu.async_copy(
            srows_v.at[pl.ds(c * _CHUNK, _CHUNK)],
            out_hbm.at[pl.ds(base + c * _CHUNK, _CHUNK)], osem))
    for cp in ocps:
        cp.wait()


def kernel(stock_ids, sector_ids, stock_table, sector_table):
    mesh = plsc.VectorSubcoreMesh(core_axis_name="c", subcore_axis_name="s")
    run = functools.partial(
        pl.kernel,
        mesh=mesh,
        out_type=jax.ShapeDtypeStruct((BATCH, D_MODEL), jnp.float32),
        scratch_types=[
            pltpu.VMEM((_NCH, _CHUNK), jnp.int32),
            pltpu.VMEM((_NCH, _CHUNK), jnp.int32),
            pltpu.VMEM_SHARED((N_SECTORS, D_MODEL), jnp.float32),
            pltpu.VMEM((_BPW, D_MODEL), jnp.float32),
            pltpu.VMEM((_BPW, D_MODEL), jnp.float32),
            pltpu.SemaphoreType.DMA,
            pltpu.SemaphoreType.DMA,
            pltpu.SemaphoreType.DMA,
            pltpu.SemaphoreType.DMA,
        ],
        compiler_params=pltpu.CompilerParams(use_tc_tiling_on_sc=False),
    )(_emb_body)
    sids = stock_ids.astype(jnp.int32).reshape(_NW * _NCH, _CHUNK)
    eids = sector_ids.astype(jnp.int32).reshape(_NW * _NCH, _CHUNK)
    return run(sids, eids, stock_table, sector_table)


# speedup vs baseline: 1.3525x; 1.0053x over previous
import functools

import jax
import jax.numpy as jnp
from jax import lax
from jax.experimental import pallas as pl
from jax.experimental.pallas import tpu as pltpu
from jax.experimental.pallas import tpu_sc as plsc

N_STOCKS = 100000
N_SECTORS = 20
D_MODEL = 64
BATCH = 16384

_NC = 2
_NS = 16
_NW = _NC * _NS
_BPW = BATCH // _NW
_CHUNK = 128
_NCH = _BPW // _CHUNK


def _emb_body(stock_ids_hbm, sector_ids_hbm, stock_tab_hbm, sector_tab_hbm,
              out_hbm, sidx_v, eidx_v, sec_v, srows_v, erows_v,
              isem, gsem, esem, osem):
    wid = lax.axis_index("s") * _NC + lax.axis_index("c")
    base = wid * _BPW

    sub = lax.axis_index("s")

    sicps = [pltpu.async_copy(
        stock_ids_hbm.at[pl.ds(base + c * _CHUNK, _CHUNK)],
        sidx_v.at[c], isem) for c in range(_NCH)]
    eicps = [pltpu.async_copy(
        sector_ids_hbm.at[pl.ds(base + c * _CHUNK, _CHUNK)],
        eidx_v.at[c], isem) for c in range(_NCH)]

    @pl.when(sub == 0)
    def _():
        pltpu.sync_copy(sector_tab_hbm, sec_v)

    gcps = []
    for c in range(_NCH):
        sicps[c].wait()
        gcps.append(pltpu.async_copy(
            stock_tab_hbm.at[sidx_v.at[c]],
            srows_v.at[pl.ds(c * _CHUNK, _CHUNK)], gsem))
    for cp in eicps:
        cp.wait()
    plsc.subcore_barrier()
    ecps = [pltpu.async_copy(sec_v.at[eidx_v.at[c]],
                             erows_v.at[pl.ds(c * _CHUNK, _CHUNK)], esem)
            for c in range(_NCH)]

    ocps = []
    for c in range(_NCH):
        gcps[c].wait()
        ecps[c].wait()

        def body(b, carry):
            for j in range(D_MODEL // 16):
                sl = pl.ds(j * 16, 16)
                srows_v[b, sl] = srows_v[b, sl] + erows_v[b, sl]
            return carry

        lax.fori_loop(c * _CHUNK, (c + 1) * _CHUNK, body, 0)
        ocps.append(pltpu.async_copy(
            srows_v.at[pl.ds(c * _CHUNK, _CHUNK)],
            out_hbm.at[pl.ds(base + c * _CHUNK, _CHUNK)], osem))
    for cp in ocps:
        cp.wait()


def kernel(stock_ids, sector_ids, stock_table, sector_table):
    mesh = plsc.VectorSubcoreMesh(core_axis_name="c", subcore_axis_name="s")
    run = functools.partial(
        pl.kernel,
        mesh=mesh,
        out_type=jax.ShapeDtypeStruct((BATCH, D_MODEL), jnp.float32),
        scratch_types=[
            pltpu.VMEM((_NCH, _CHUNK), jnp.int32),
            pltpu.VMEM((_NCH, _CHUNK), jnp.int32),
            pltpu.VMEM_SHARED((N_SECTORS, D_MODEL), jnp.float32),
            pltpu.VMEM((_BPW, D_MODEL), jnp.float32),
            pltpu.VMEM((_BPW, D_MODEL), jnp.float32),
            pltpu.SemaphoreType.DMA,
            pltpu.SemaphoreType.DMA,
            pltpu.SemaphoreType.DMA,
            pltpu.SemaphoreType.DMA,
        ],
        compiler_params=pltpu.CompilerParams(use_tc_tiling_on_sc=False),
    )(_emb_body)
    return run(stock_ids.astype(jnp.int32), sector_ids.astype(jnp.int32),
               stock_table, sector_table)


# speedup vs baseline: 1.5433x; 1.1411x over previous
import functools

import jax
import jax.numpy as jnp
from jax import lax
from jax.experimental import pallas as pl
from jax.experimental.pallas import tpu as pltpu
from jax.experimental.pallas import tpu_sc as plsc

N_STOCKS = 100000
N_SECTORS = 20
D_MODEL = 64
BATCH = 16384

_NC = 2
_NS = 16
_NW = _NC * _NS
_BPW = BATCH // _NW
_CHUNK = 128
_NCH = _BPW // _CHUNK


def _emb_body(stock_ids_hbm, sector_ids_hbm, stock_tab_hbm, sector_tab_hbm,
              out_hbm, sidx_v, eidx_v, sec_v, srows_v, isem, gsem, osem):
    wid = lax.axis_index("s") * _NC + lax.axis_index("c")
    base = wid * _BPW

    cp_si = pltpu.async_copy(stock_ids_hbm.at[pl.ds(base, _BPW)],
                             sidx_v.at[pl.ds(0, _BPW)], isem)
    cp_ei = pltpu.async_copy(sector_ids_hbm.at[pl.ds(base, _BPW)],
                             eidx_v.at[pl.ds(0, _BPW)], isem)
    cp_st = pltpu.async_copy(sector_tab_hbm, sec_v, isem)
    cp_si.wait()

    def fire(b, carry):
        sid = sidx_v[pl.ds(b, 16)][0]
        pltpu.async_copy(stock_tab_hbm.at[sid], srows_v.at[b], gsem)
        return carry

    ocps = []
    for c in range(_NCH):
        lax.fori_loop(c * _CHUNK, (c + 1) * _CHUNK, fire, 0)
        pltpu.make_async_copy(
            stock_tab_hbm.at[pl.ds(0, _CHUNK)],
            srows_v.at[pl.ds(c * _CHUNK, _CHUNK)], gsem).wait()
        if c == 0:
            cp_ei.wait()
            cp_st.wait()

        def add(b, carry):
            eid = eidx_v[pl.ds(b, 16)][0]
            for j in range(D_MODEL // 16):
                sl = pl.ds(j * 16, 16)
                srows_v[b, sl] = srows_v[b, sl] + sec_v[eid, sl]
            return carry

        lax.fori_loop(c * _CHUNK, (c + 1) * _CHUNK, add, 0)
        ocps.append(pltpu.async_copy(
            srows_v.at[pl.ds(c * _CHUNK, _CHUNK)],
            out_hbm.at[pl.ds(base + c * _CHUNK, _CHUNK)], osem))
    for cp in ocps:
        cp.wait()


def kernel(stock_ids, sector_ids, stock_table, sector_table):
    mesh = plsc.VectorSubcoreMesh(core_axis_name="c", subcore_axis_name="s")
    run = functools.partial(
        pl.kernel,
        mesh=mesh,
        out_type=jax.ShapeDtypeStruct((BATCH, D_MODEL), jnp.float32),
        scratch_types=[
            pltpu.VMEM((_BPW + 16,), jnp.int32),
            pltpu.VMEM((_BPW + 16,), jnp.int32),
            pltpu.VMEM((N_SECTORS, D_MODEL), jnp.float32),
            pltpu.VMEM((_BPW, D_MODEL), jnp.float32),
            pltpu.SemaphoreType.DMA,
            pltpu.SemaphoreType.DMA,
            pltpu.SemaphoreType.DMA,
        ],
    )(_emb_body)
    return run(stock_ids.astype(jnp.int32), sector_ids.astype(jnp.int32),
               stock_table, sector_table)


# speedup vs baseline: 1.7806x; 1.1537x over previous
import functools

import jax
import jax.numpy as jnp
from jax import lax
from jax.experimental import pallas as pl
from jax.experimental.pallas import tpu as pltpu
from jax.experimental.pallas import tpu_sc as plsc

N_STOCKS = 100000
N_SECTORS = 20
D_MODEL = 64
BATCH = 16384

_NC = 2
_NS = 16
_NW = _NC * _NS
_BPW = BATCH // _NW
_CHUNK = 128
_NCH = _BPW // _CHUNK


def _emb_body(stock_ids_hbm, sector_ids_hbm, stock_tab_hbm, sector_tab_hbm,
              out_hbm, sidx_v, eidx_v, sec_v, srows_v, isem, gsem, osem2,
              osem):
    wid = lax.axis_index("s") * _NC + lax.axis_index("c")
    base = wid * _BPW

    cp_si = pltpu.async_copy(stock_ids_hbm.at[pl.ds(base, _BPW)],
                             sidx_v.at[pl.ds(0, _BPW)], isem)
    cp_ei = pltpu.async_copy(sector_ids_hbm.at[pl.ds(base, _BPW)],
                             eidx_v.at[pl.ds(0, _BPW)], isem)
    cp_st = pltpu.async_copy(sector_tab_hbm, sec_v, isem)
    cp_si.wait()

    def fire_chunk(c, sem):
        def fire16(i, carry):
            b0 = i * 16
            ids16 = sidx_v[pl.ds(b0, 16)]
            for k in range(16):
                pltpu.async_copy(stock_tab_hbm.at[ids16[k]],
                                 srows_v.at[b0 + k], sem)
            return carry
        lax.fori_loop(c * (_CHUNK // 16), (c + 1) * (_CHUNK // 16),
                      fire16, 0)

    def drain_chunk(c, sem):
        pltpu.make_async_copy(
            stock_tab_hbm.at[pl.ds(0, _CHUNK)],
            srows_v.at[pl.ds(c * _CHUNK, _CHUNK)], sem).wait()

    def add_chunk(c):
        def add16(i, carry):
            b0 = i * 16
            eids16 = eidx_v[pl.ds(b0, 16)]
            for k in range(16):
                eid = eids16[k]
                for j in range(D_MODEL // 16):
                    sl = pl.ds(j * 16, 16)
                    srows_v[b0 + k, sl] = (srows_v[b0 + k, sl]
                                           + sec_v[eid, sl])
            return carry
        lax.fori_loop(c * (_CHUNK // 16), (c + 1) * (_CHUNK // 16),
                      add16, 0)

    gsems = [gsem, osem2]
    fire_chunk(0, gsems[0])
    cp_ei.wait()
    cp_st.wait()
    ocps = []
    for c in range(_NCH):
        if c + 1 < _NCH:
            fire_chunk(c + 1, gsems[(c + 1) % 2])
        drain_chunk(c, gsems[c % 2])
        add_chunk(c)
        ocps.append(pltpu.async_copy(
            srows_v.at[pl.ds(c * _CHUNK, _CHUNK)],
            out_hbm.at[pl.ds(base + c * _CHUNK, _CHUNK)], osem))
    for cp in ocps:
        cp.wait()


def kernel(stock_ids, sector_ids, stock_table, sector_table):
    mesh = plsc.VectorSubcoreMesh(core_axis_name="c", subcore_axis_name="s")
    run = functools.partial(
        pl.kernel,
        mesh=mesh,
        out_type=jax.ShapeDtypeStruct((BATCH, D_MODEL), jnp.float32),
        scratch_types=[
            pltpu.VMEM((_BPW + 16,), jnp.int32),
            pltpu.VMEM((_BPW + 16,), jnp.int32),
            pltpu.VMEM((N_SECTORS, D_MODEL), jnp.float32),
            pltpu.VMEM((_BPW, D_MODEL), jnp.float32),
            pltpu.SemaphoreType.DMA,
            pltpu.SemaphoreType.DMA,
            pltpu.SemaphoreType.DMA,
            pltpu.SemaphoreType.DMA,
        ],
    )(_emb_body)
    return run(stock_ids.astype(jnp.int32), sector_ids.astype(jnp.int32),
               stock_table, sector_table)


# speedup vs baseline: 2.0413x; 1.1464x over previous
import functools

import jax
import jax.numpy as jnp
from jax import lax
from jax.experimental import pallas as pl
from jax.experimental.pallas import tpu as pltpu
from jax.experimental.pallas import tpu_sc as plsc

N_STOCKS = 100000
N_SECTORS = 20
D_MODEL = 64
BATCH = 16384

_NC = 2
_NS = 16
_NW = _NC * _NS
_DPW = D_MODEL // _NW
_BBLK = 2048
_NBLK = BATCH // _BBLK


def _emb_body(stock_ids_hbm, sector_ids_hbm, stock_tabT_hbm, sector_tabT_hbm,
              outT_hbm, row_v, secrow_v, sidx_v, eidx_v, out_v,
              isem, rsem, osem):
    wid = lax.axis_index("s") * _NC + lax.axis_index("c")

    for t in range(_DPW):
        d = wid * _DPW + t
        cp_row = pltpu.async_copy(stock_tabT_hbm.at[d], row_v, rsem)
        cp_sec = pltpu.async_copy(sector_tabT_hbm.at[d], secrow_v, rsem)
        ocps = []
        for blk in range(_NBLK):
            cp_si = pltpu.async_copy(
                stock_ids_hbm.at[pl.ds(blk * _BBLK, _BBLK)],
                sidx_v.at[blk % 2], isem)
            cp_ei = pltpu.async_copy(
                sector_ids_hbm.at[pl.ds(blk * _BBLK, _BBLK)],
                eidx_v.at[blk % 2], isem)
            if blk == 0:
                cp_row.wait()
                cp_sec.wait()
            cp_si.wait()
            cp_ei.wait()

            def gather16(i, carry):
                b0 = i * 16
                ids16 = sidx_v[blk % 2, pl.ds(b0, 16)]
                eids16 = eidx_v[blk % 2, pl.ds(b0, 16)]
                svals = plsc.load_gather(row_v, [ids16])
                evals = plsc.load_gather(secrow_v, [eids16])
                out_v[blk % 2, pl.ds(b0, 16)] = svals + evals
                return carry

            lax.fori_loop(0, _BBLK // 16, gather16, 0)
            if blk >= 2:
                ocps[blk - 2].wait()
            ocps.append(pltpu.async_copy(
                out_v.at[blk % 2],
                outT_hbm.at[d, pl.ds(blk * _BBLK, _BBLK)], osem))
        ocps[-2].wait()
        ocps[-1].wait()


def kernel(stock_ids, sector_ids, stock_table, sector_table):
    mesh = plsc.VectorSubcoreMesh(core_axis_name="c", subcore_axis_name="s")
    run = functools.partial(
        pl.kernel,
        mesh=mesh,
        out_type=jax.ShapeDtypeStruct((D_MODEL, BATCH), jnp.float32),
        scratch_types=[
            pltpu.VMEM((N_STOCKS,), jnp.float32),
            pltpu.VMEM((N_SECTORS,), jnp.float32),
            pltpu.VMEM((2, _BBLK), jnp.int32),
            pltpu.VMEM((2, _BBLK), jnp.int32),
            pltpu.VMEM((2, _BBLK), jnp.float32),
            pltpu.SemaphoreType.DMA,
            pltpu.SemaphoreType.DMA,
            pltpu.SemaphoreType.DMA,
        ],
        compiler_params=pltpu.CompilerParams(needs_layout_passes=False),
    )(_emb_body)
    outT = run(stock_ids.astype(jnp.int32), sector_ids.astype(jnp.int32),
               stock_table.T, sector_table.T)
    return outT.T


# speedup vs baseline: 2.3322x; 1.1425x over previous
import functools

import jax
import jax.numpy as jnp
from jax import lax
from jax.experimental import pallas as pl
from jax.experimental.pallas import tpu as pltpu
from jax.experimental.pallas import tpu_sc as plsc

N_STOCKS = 100000
N_SECTORS = 20
D_MODEL = 64
BATCH = 16384

_NC = 2
_NS = 16
_NW = _NC * _NS
_DPW = D_MODEL // _NW
_BBLK = 2048
_NBLK = BATCH // _BBLK


def _emb_body(stock_ids_hbm, sector_ids_hbm, stock_tabT_hbm, sector_tabT_hbm,
              outT_hbm, row_v, secrow_v, sidx_v, eidx_v, out_v,
              isem, isem2, rsem, osem):
    wid = lax.axis_index("s") * _NC + lax.axis_index("c")

    isems = [isem, isem2]

    def fire_ids(blk):
        pltpu.async_copy(stock_ids_hbm.at[pl.ds(blk * _BBLK, _BBLK)],
                         sidx_v.at[blk % 2], isems[blk % 2])
        pltpu.async_copy(sector_ids_hbm.at[pl.ds(blk * _BBLK, _BBLK)],
                         eidx_v.at[blk % 2], isems[blk % 2])

    def drain_ids(blk):
        pltpu.make_async_copy(stock_ids_hbm.at[pl.ds(0, _BBLK)],
                              sidx_v.at[blk % 2], isems[blk % 2]).wait()
        pltpu.make_async_copy(sector_ids_hbm.at[pl.ds(0, _BBLK)],
                              eidx_v.at[blk % 2], isems[blk % 2]).wait()

    cp_row = pltpu.async_copy(stock_tabT_hbm.at[wid * _DPW], row_v, rsem)
    cp_sec = pltpu.async_copy(sector_tabT_hbm.at[wid * _DPW], secrow_v,
                              rsem)
    for t in range(_DPW):
        d = wid * _DPW + t
        fire_ids(0)
        ocps = []
        for blk in range(_NBLK):
            if blk + 1 < _NBLK:
                fire_ids(blk + 1)
            if blk == 0:
                cp_row.wait()
                cp_sec.wait()
            drain_ids(blk)

            def gather32(i, carry):
                for u in range(2):
                    b0 = i * 32 + u * 16
                    ids16 = sidx_v[blk % 2, pl.ds(b0, 16)]
                    eids16 = eidx_v[blk % 2, pl.ds(b0, 16)]
                    svals = plsc.load_gather(row_v, [ids16])
                    evals = plsc.load_gather(secrow_v, [eids16])
                    out_v[blk % 2, pl.ds(b0, 16)] = svals + evals
                return carry

            lax.fori_loop(0, _BBLK // 32, gather32, 0)
            if blk >= 2:
                ocps[blk - 2].wait()
            ocps.append(pltpu.async_copy(
                out_v.at[blk % 2],
                outT_hbm.at[d, pl.ds(blk * _BBLK, _BBLK)], osem))
        if t + 1 < _DPW:
            cp_row = pltpu.async_copy(stock_tabT_hbm.at[d + 1], row_v, rsem)
            cp_sec = pltpu.async_copy(sector_tabT_hbm.at[d + 1], secrow_v,
                                      rsem)
        ocps[-2].wait()
        ocps[-1].wait()


def kernel(stock_ids, sector_ids, stock_table, sector_table):
    mesh = plsc.VectorSubcoreMesh(core_axis_name="c", subcore_axis_name="s")
    run = functools.partial(
        pl.kernel,
        mesh=mesh,
        out_type=jax.ShapeDtypeStruct((D_MODEL, BATCH), jnp.float32),
        scratch_types=[
            pltpu.VMEM((N_STOCKS,), jnp.float32),
            pltpu.VMEM((N_SECTORS,), jnp.float32),
            pltpu.VMEM((2, _BBLK), jnp.int32),
            pltpu.VMEM((2, _BBLK), jnp.int32),
            pltpu.VMEM((2, _BBLK), jnp.float32),
            pltpu.SemaphoreType.DMA,
            pltpu.SemaphoreType.DMA,
            pltpu.SemaphoreType.DMA,
            pltpu.SemaphoreType.DMA,
        ],
        compiler_params=pltpu.CompilerParams(needs_layout_passes=False),
    )(_emb_body)
    outT = run(stock_ids.astype(jnp.int32), sector_ids.astype(jnp.int32),
               stock_table.T, sector_table.T)
    return outT.T
